# baseline (speedup 1.0000x reference)
import functools

import jax
import jax.numpy as jnp
from jax.experimental import pallas as pl

N = 10000
E = 320000
D = 128
ED = 16
EPS = 1e-5


def _pre_matmul_kernel(hc_ref, w_ref, out_ref):
    out_ref[...] = jnp.dot(hc_ref[...], w_ref[...],
                           preferred_element_type=jnp.float32)


def _edge_c_kernel(e_ref, w_ref, b_ref, out_ref):
    out_ref[...] = jnp.dot(e_ref[...], w_ref[...],
                           preferred_element_type=jnp.float32) + b_ref[...]


def _post_kernel(hc_ref, agg_ref, w1_ref, w2_ref, b_ref, snorm_ref,
                 gam_ref, bet_ref, mu_ref, var_ref, h_ref, p_ref,
                 ho_ref, po_ref):
    h3 = jnp.dot(hc_ref[...], w1_ref[...], preferred_element_type=jnp.float32)
    h3 += jnp.dot(agg_ref[...], w2_ref[...], preferred_element_type=jnp.float32)
    h3 += b_ref[...]
    h3 = h3 * snorm_ref[...]
    h3 = (h3 - mu_ref[...]) * jax.lax.rsqrt(var_ref[...] + EPS) * gam_ref[...] + bet_ref[...]
    h3 = jnp.maximum(h3, 0.0)
    ho_ref[...] = h_ref[...] + h3
    po_ref[...] = p_ref[...] + jnp.tanh(h3)


def kernel(h, p, e, snorm_n, edge_index, W_pre_h, b_pre_h, W_pre_p, b_pre_p,
           W_post_h, b_post_h, W_post_p, b_post_p, bn_gamma, bn_beta,
           bn_mean, bn_var):
    hc = jnp.concatenate([h, p], axis=1)
    W12 = jnp.concatenate([W_pre_h[:2 * D], W_pre_h[2 * D:4 * D]], axis=1)
    W3 = W_pre_h[4 * D:]

    AB = pl.pallas_call(
        _pre_matmul_kernel,
        out_shape=jax.ShapeDtypeStruct((N, 2 * D), jnp.float32),
        grid=(10,),
        in_specs=[pl.BlockSpec((N // 10, 2 * D), lambda i: (i, 0)),
                  pl.BlockSpec((2 * D, 2 * D), lambda i: (0, 0))],
        out_specs=pl.BlockSpec((N // 10, 2 * D), lambda i: (i, 0)),
    )(hc, W12)

    C = pl.pallas_call(
        _edge_c_kernel,
        out_shape=jax.ShapeDtypeStruct((E, D), jnp.float32),
        grid=(16,),
        in_specs=[pl.BlockSpec((E // 16, ED), lambda i: (i, 0)),
                  pl.BlockSpec((ED, D), lambda i: (0, 0)),
                  pl.BlockSpec((D,), lambda i: (0,))],
        out_specs=pl.BlockSpec((E // 16, D), lambda i: (i, 0)),
    )(e, W3, b_pre_h)

    src = edge_index[0]
    dst = edge_index[1]
    msgs = AB[src, :D] + AB[dst, D:] + C

    ones = jnp.ones((E,), jnp.float32)
    deg = jax.ops.segment_sum(ones, dst, num_segments=N)
    safe = jnp.maximum(deg, 1.0)[:, None]
    s = jax.ops.segment_sum(msgs, dst, num_segments=N)
    mean = s / safe
    sq = jax.ops.segment_sum(msgs * msgs, dst, num_segments=N) / safe
    var = jnp.maximum(sq - mean * mean, 0.0)
    std = jnp.sqrt(var + EPS)
    pos = deg[:, None] > 0
    mx = jnp.where(pos, jax.ops.segment_max(msgs, dst, num_segments=N), 0.0)
    mn = jnp.where(pos, -jax.ops.segment_max(-msgs, dst, num_segments=N), 0.0)
    agg = jnp.concatenate([mean, mx, mn, std], axis=1)

    ho, po = pl.pallas_call(
        _post_kernel,
        out_shape=(jax.ShapeDtypeStruct((N, D), jnp.float32),
                   jax.ShapeDtypeStruct((N, D), jnp.float32)),
        grid=(10,),
        in_specs=[pl.BlockSpec((N // 10, 2 * D), lambda i: (i, 0)),
                  pl.BlockSpec((N // 10, 4 * D), lambda i: (i, 0)),
                  pl.BlockSpec((2 * D, D), lambda i: (0, 0)),
                  pl.BlockSpec((4 * D, D), lambda i: (0, 0)),
                  pl.BlockSpec((D,), lambda i: (0,)),
                  pl.BlockSpec((N // 10, 1), lambda i: (i, 0)),
                  pl.BlockSpec((D,), lambda i: (0,)),
                  pl.BlockSpec((D,), lambda i: (0,)),
                  pl.BlockSpec((D,), lambda i: (0,)),
                  pl.BlockSpec((D,), lambda i: (0,)),
                  pl.BlockSpec((N // 10, D), lambda i: (i, 0)),
                  pl.BlockSpec((N // 10, D), lambda i: (i, 0))],
        out_specs=(pl.BlockSpec((N // 10, D), lambda i: (i, 0)),
                   pl.BlockSpec((N // 10, D), lambda i: (i, 0))),
    )(hc, agg, W_post_h[:2 * D], W_post_h[2 * D:], b_post_h, snorm_n,
      bn_gamma, bn_beta, bn_mean, bn_var, h, p)
    return (ho, po)


# speedup vs baseline: 73.8021x; 73.8021x over previous
import functools

import jax
import jax.numpy as jnp
from jax import lax
from jax.experimental import pallas as pl
from jax.experimental.pallas import tpu as pltpu
from jax.experimental.pallas import tpu_sc as plsc

N = 10000
E = 320000
D = 128
ED = 16
EPS = 1e-5

NW = 32
NPW = 320
NPAD = NW * NPW
HALF = 160
NLOC = 168
CHUNK = 2000
GB = 32
BIG = 3.0e38

_GDN = lax.GatherDimensionNumbers(offset_dims=(), collapsed_slice_dims=(0,),
                                  start_index_map=(0,))


def _vgather(x, idx):
    return lax.gather(x, idx[:, None], _GDN, (1,),
                      mode=lax.GatherScatterMode.PROMISE_IN_BOUNDS)


def _pre_ab_kernel(hc_ref, w1_ref, w2_ref, a_ref, b_ref):
    a_ref[...] = jnp.dot(hc_ref[...], w1_ref[...],
                         preferred_element_type=jnp.float32)
    b_ref[...] = jnp.dot(hc_ref[...], w2_ref[...],
                         preferred_element_type=jnp.float32)


def _edge_c_kernel(e_ref, w_ref, b_ref, out_ref):
    out_ref[...] = jnp.dot(e_ref[...], w_ref[...],
                           preferred_element_type=jnp.float32) + b_ref[...]


def _sc_agg_body(a_hbm, c_hbm, src_hbm, dst_hbm,
                 sm_hbm, sq_hbm, mx_hbm, mn_hbm, deg_hbm,
                 dstbuf, ids, lds, idg, ldsg, srcb, abuf, cbuf,
                 sm, sq, mx, mn, deg, sem_a, sem_c):
    c_id = lax.axis_index("c")
    s_id = lax.axis_index("s")
    wid = c_id * 16 + s_id
    lanes = lax.iota(jnp.int32, 16)
    ones16 = jnp.ones((16,), jnp.float32)

    for rnd in range(2):
        lo = wid * NPW + rnd * HALF

        def init_accs(i, _):
            for k in range(D // 16):
                sl = pl.ds(k * 16, 16)
                sm[i, sl] = jnp.zeros((16,), jnp.float32)
                sq[i, sl] = jnp.zeros((16,), jnp.float32)
                mx[i, sl] = jnp.full((16,), -BIG, jnp.float32)
                mn[i, sl] = jnp.full((16,), BIG, jnp.float32)
            deg[i, pl.ds(0, 16)] = jnp.zeros((16,), jnp.float32)
            return 0
        lax.fori_loop(0, NLOC, init_accs, 0)

        def chunk_body(ci, _):
            pltpu.sync_copy(dst_hbm.at[pl.ds(ci * CHUNK, CHUNK)], dstbuf)

            def scan_body(j, cnt):
                d = dstbuf[pl.ds(j * 16, 16)]
                dl = d - lo
                m = (dl >= 0) & (dl < HALF)
                p = jnp.where(m, 1, 0).astype(jnp.int32)
                for sh in (1, 2, 4, 8):
                    shifted = _vgather(p, jnp.maximum(lanes - sh, 0))
                    p = p + jnp.where(lanes >= sh, shifted, 0)
                tot = p[15]
                r1 = lanes + 1
                b = jnp.zeros((16,), jnp.int32)
                for sh in (8, 4, 2, 1):
                    idx = jnp.minimum(b + (sh - 1), 15)
                    pv = _vgather(p, idx)
                    b = jnp.where(pv < r1, b + sh, b)
                perm = jnp.minimum(b, 15)
                ids[pl.ds(cnt, 16)] = perm + (ci * CHUNK + j * 16)
                lds[pl.ds(cnt, 16)] = _vgather(dl, perm)
                return cnt + tot
            cnt = lax.fori_loop(0, CHUNK // 16, scan_body, 0)

            pad_id = lanes + (wid * 13 % 256)
            pad_ld = HALF + lax.rem(lanes, 8)
            ids[pl.ds(cnt, 16)] = pad_id
            ids[pl.ds(cnt + 16, 16)] = pad_id
            lds[pl.ds(cnt, 16)] = pad_ld
            lds[pl.ds(cnt + 16, 16)] = pad_ld
            ng = (cnt + GB - 1) // GB

            def group_body(g, _):
                for t in range(GB // 16):
                    sl = pl.ds(t * 16, 16)
                    idg[sl] = ids[pl.ds(g * GB + t * 16, 16)]
                    ldsg[sl] = lds[pl.ds(g * GB + t * 16, 16)]
                cpc = pltpu.async_copy(c_hbm.at[idg], cbuf, sem_c)
                pltpu.async_copy(src_hbm.at[idg], srcb, sem_a).wait()
                pltpu.async_copy(a_hbm.at[srcb], abuf, sem_a).wait()
                cpc.wait()

                for t in range(GB // 16):
                    lvec = ldsg[pl.ds(t * 16, 16)]
                    for e16 in range(16):
                        e = t * 16 + e16
                        l = lvec[e16]
                        deg[l, pl.ds(0, 16)] += ones16
                        for k in range(D // 16):
                            sl = pl.ds(k * 16, 16)
                            u = abuf[e, sl] + cbuf[e, sl]
                            plsc.addupdate(sm.at[l, sl], u)
                            plsc.addupdate(sq.at[l, sl], u * u)
                            mx[l, sl] = jnp.maximum(mx[l, sl], u)
                            mn[l, sl] = jnp.minimum(mn[l, sl], u)
                return 0
            lax.fori_loop(0, ng, group_body, 0)
            return 0
        lax.fori_loop(0, E // CHUNK, chunk_body, 0)

        pltpu.sync_copy(sm.at[pl.ds(0, HALF)], sm_hbm.at[pl.ds(lo, HALF)])
        pltpu.sync_copy(sq.at[pl.ds(0, HALF)], sq_hbm.at[pl.ds(lo, HALF)])
        pltpu.sync_copy(mx.at[pl.ds(0, HALF)], mx_hbm.at[pl.ds(lo, HALF)])
        pltpu.sync_copy(mn.at[pl.ds(0, HALF)], mn_hbm.at[pl.ds(lo, HALF)])
        pltpu.sync_copy(deg.at[pl.ds(0, HALF)], deg_hbm.at[pl.ds(lo, HALF)])


def _sc_aggregate(A, C, src, dst):
    mesh = plsc.VectorSubcoreMesh(core_axis_name="c", subcore_axis_name="s")
    f32 = jnp.float32
    kern = functools.partial(
        pl.kernel, mesh=mesh,
        out_type=[jax.ShapeDtypeStruct((NPAD, D), f32),
                  jax.ShapeDtypeStruct((NPAD, D), f32),
                  jax.ShapeDtypeStruct((NPAD, D), f32),
                  jax.ShapeDtypeStruct((NPAD, D), f32),
                  jax.ShapeDtypeStruct((NPAD, 16), f32)],
        scratch_types=[
            pltpu.VMEM((CHUNK,), jnp.int32),
            pltpu.VMEM((CHUNK + 304,), jnp.int32),
            pltpu.VMEM((CHUNK + 304,), jnp.int32),
            pltpu.VMEM((GB,), jnp.int32),
            pltpu.VMEM((GB,), jnp.int32),
            pltpu.VMEM((GB,), jnp.int32),
            pltpu.VMEM((GB, D), f32),
            pltpu.VMEM((GB, D), f32),
            pltpu.VMEM((NLOC, D), f32),
            pltpu.VMEM((NLOC, D), f32),
            pltpu.VMEM((NLOC, D), f32),
            pltpu.VMEM((NLOC, D), f32),
            pltpu.VMEM((NLOC, 16), f32),
            pltpu.SemaphoreType.DMA,
            pltpu.SemaphoreType.DMA,
        ],
    )(_sc_agg_body)
    return kern(A, C, src, dst)


def _post_kernel(hc_ref, b_ref, sm_ref, sq_ref, mx_ref, mn_ref, degb_ref,
                 whc_ref, w1_ref, w2_ref, w3_ref, w4_ref, bb_ref,
                 snorm_ref, gam_ref, bet_ref, mu_ref, var_ref,
                 h_ref, p_ref, ho_ref, po_ref):
    B = b_ref[...]
    sumU = sm_ref[...]
    sqU = sq_ref[...]
    deg = degb_ref[...][:, :1]
    safe = jnp.maximum(deg, 1.0)
    mean = (sumU + deg * B) / safe
    meansq = (sqU + 2.0 * B * sumU + deg * (B * B)) / safe
    var = jnp.maximum(meansq - mean * mean, 0.0)
    std = jnp.sqrt(var + EPS)
    pos = deg > 0.0
    mxf = jnp.where(pos, mx_ref[...] + B, 0.0)
    mnf = jnp.where(pos, mn_ref[...] + B, 0.0)
    h3 = jnp.dot(hc_ref[...], whc_ref[...], preferred_element_type=jnp.float32)
    h3 += jnp.dot(mean, w1_ref[...], preferred_element_type=jnp.float32)
    h3 += jnp.dot(mxf, w2_ref[...], preferred_element_type=jnp.float32)
    h3 += jnp.dot(mnf, w3_ref[...], preferred_element_type=jnp.float32)
    h3 += jnp.dot(std, w4_ref[...], preferred_element_type=jnp.float32)
    h3 += bb_ref[...]
    h3 = h3 * snorm_ref[...]
    h3 = (h3 - mu_ref[...]) * lax.rsqrt(var_ref[...] + EPS) * gam_ref[...] + bet_ref[...]
    h3 = jnp.maximum(h3, 0.0)
    ho_ref[...] = h_ref[...] + h3
    po_ref[...] = p_ref[...] + jnp.tanh(h3)


def kernel(h, p, e, snorm_n, edge_index, W_pre_h, b_pre_h, W_pre_p, b_pre_p,
           W_post_h, b_post_h, W_post_p, b_post_p, bn_gamma, bn_beta,
           bn_mean, bn_var):
    hc = jnp.concatenate([h, p], axis=1)
    W1 = W_pre_h[:2 * D]
    W2 = W_pre_h[2 * D:4 * D]
    W3 = W_pre_h[4 * D:]

    A, B = pl.pallas_call(
        _pre_ab_kernel,
        out_shape=(jax.ShapeDtypeStruct((N, D), jnp.float32),
                   jax.ShapeDtypeStruct((N, D), jnp.float32)),
        grid=(10,),
        in_specs=[pl.BlockSpec((N // 10, 2 * D), lambda i: (i, 0)),
                  pl.BlockSpec((2 * D, D), lambda i: (0, 0)),
                  pl.BlockSpec((2 * D, D), lambda i: (0, 0))],
        out_specs=(pl.BlockSpec((N // 10, D), lambda i: (i, 0)),
                   pl.BlockSpec((N // 10, D), lambda i: (i, 0))),
    )(hc, W1, W2)

    C = pl.pallas_call(
        _edge_c_kernel,
        out_shape=jax.ShapeDtypeStruct((E, D), jnp.float32),
        grid=(16,),
        in_specs=[pl.BlockSpec((E // 16, ED), lambda i: (i, 0)),
                  pl.BlockSpec((ED, D), lambda i: (0, 0)),
                  pl.BlockSpec((D,), lambda i: (0,))],
        out_specs=pl.BlockSpec((E // 16, D), lambda i: (i, 0)),
    )(e, W3, b_pre_h)

    src = edge_index[0]
    dst = edge_index[1]
    sm, sq, mx, mn, degb = _sc_aggregate(A, C, src, dst)

    ho, po = pl.pallas_call(
        _post_kernel,
        out_shape=(jax.ShapeDtypeStruct((N, D), jnp.float32),
                   jax.ShapeDtypeStruct((N, D), jnp.float32)),
        grid=(10,),
        in_specs=[pl.BlockSpec((N // 10, 2 * D), lambda i: (i, 0)),
                  pl.BlockSpec((N // 10, D), lambda i: (i, 0)),
                  pl.BlockSpec((N // 10, D), lambda i: (i, 0)),
                  pl.BlockSpec((N // 10, D), lambda i: (i, 0)),
                  pl.BlockSpec((N // 10, D), lambda i: (i, 0)),
                  pl.BlockSpec((N // 10, D), lambda i: (i, 0)),
                  pl.BlockSpec((N // 10, 16), lambda i: (i, 0)),
                  pl.BlockSpec((2 * D, D), lambda i: (0, 0)),
                  pl.BlockSpec((D, D), lambda i: (0, 0)),
                  pl.BlockSpec((D, D), lambda i: (0, 0)),
                  pl.BlockSpec((D, D), lambda i: (0, 0)),
                  pl.BlockSpec((D, D), lambda i: (0, 0)),
                  pl.BlockSpec((D,), lambda i: (0,)),
                  pl.BlockSpec((N // 10, 1), lambda i: (i, 0)),
                  pl.BlockSpec((D,), lambda i: (0,)),
                  pl.BlockSpec((D,), lambda i: (0,)),
                  pl.BlockSpec((D,), lambda i: (0,)),
                  pl.BlockSpec((D,), lambda i: (0,)),
                  pl.BlockSpec((N // 10, D), lambda i: (i, 0)),
                  pl.BlockSpec((N // 10, D), lambda i: (i, 0))],
        out_specs=(pl.BlockSpec((N // 10, D), lambda i: (i, 0)),
                   pl.BlockSpec((N // 10, D), lambda i: (i, 0))),
    )(hc, B, sm[:N], sq[:N], mx[:N], mn[:N], degb[:N],
      W_post_h[:2 * D], W_post_h[2 * D:3 * D], W_post_h[3 * D:4 * D],
      W_post_h[4 * D:5 * D], W_post_h[5 * D:], b_post_h, snorm_n,
      bn_gamma, bn_beta, bn_mean, bn_var, h, p)
    return (ho, po)


# speedup vs baseline: 120.7686x; 1.6364x over previous
import functools

import jax
import jax.numpy as jnp
from jax import lax
from jax.experimental import pallas as pl
from jax.experimental.pallas import tpu as pltpu
from jax.experimental.pallas import tpu_sc as plsc

N = 10000
E = 320000
D = 128
ED = 16
EPS = 1e-5

NW = 32
NPW = 320
NPAD = NW * NPW
HALF = 160
NLOC = 161
CHUNK = 800
GB = 16
LCAP = 5440
BIG = 3.0e38

_GDN = lax.GatherDimensionNumbers(offset_dims=(), collapsed_slice_dims=(0,),
                                  start_index_map=(0,))


def _vgather(x, idx):
    return lax.gather(x, idx[:, None], _GDN, (1,),
                      mode=lax.GatherScatterMode.PROMISE_IN_BOUNDS)


def _pre_ab_kernel(hc_ref, w1_ref, w2_ref, a_ref, b_ref):
    a_ref[...] = jnp.dot(hc_ref[...], w1_ref[...],
                         preferred_element_type=jnp.float32)
    b_ref[...] = jnp.dot(hc_ref[...], w2_ref[...],
                         preferred_element_type=jnp.float32)


def _edge_c_kernel(e_ref, w_ref, b_ref, out_ref):
    out_ref[...] = jnp.dot(e_ref[...], w_ref[...],
                           preferred_element_type=jnp.float32) + b_ref[...]


def _compact(vals, m, lanes):
    p = jnp.where(m, 1, 0).astype(jnp.int32)
    for sh in (1, 2, 4, 8):
        shifted = _vgather(p, jnp.maximum(lanes - sh, 0))
        p = p + jnp.where(lanes >= sh, shifted, 0)
    tot = p[15]
    r1 = lanes + 1
    b = jnp.zeros((16,), jnp.int32)
    for sh in (8, 4, 2, 1):
        idx = jnp.minimum(b + (sh - 1), 15)
        pv = _vgather(p, idx)
        b = jnp.where(pv < r1, b + sh, b)
    return jnp.minimum(b, 15), tot


def _sc_agg_body(a_hbm, c_hbm, src_hbm, dst_hbm,
                 sm_hbm, sq_hbm, mx_hbm, mn_hbm, deg_hbm,
                 dstbuf, tid, tld, idsA, idsB,
                 idg, dstg, srcb, abuf, cbuf,
                 sm, sq, mx, mn, deg, sem_a, sem_c, sem_a2, sem_c2):
    c_id = lax.axis_index("c")
    s_id = lax.axis_index("s")
    wid = c_id * 16 + s_id
    lo0 = wid * NPW
    lanes = lax.iota(jnp.int32, 16)
    ones16 = jnp.ones((16,), jnp.float32)
    sems_a = (sem_a, sem_a2)
    sems_c = (sem_c, sem_c2)

    def chunk_body(ci, carry):
        cA, cB = carry
        pltpu.sync_copy(dst_hbm.at[pl.ds(ci * CHUNK, CHUNK)], dstbuf)

        def scan_body(j, cnt):
            d = dstbuf[pl.ds(j * 16, 16)]
            dl = d - lo0
            m = (dl >= 0) & (dl < NPW)
            perm, tot = _compact(dl, m, lanes)

            @pl.when(tot > 0)
            def _():
                tid[pl.ds(cnt, 16)] = perm + (ci * CHUNK + j * 16)
                tld[pl.ds(cnt, 16)] = _vgather(dl, perm)
            return cnt + tot
        cnt = lax.fori_loop(0, CHUNK // 16, scan_body, 0)
        tid[pl.ds(cnt, 16)] = jnp.zeros((16,), jnp.int32)
        tld[pl.ds(cnt, 16)] = jnp.full((16,), 2 * NPW, jnp.int32)

        def part_body(v, carry):
            cA, cB = carry
            idv = tid[pl.ds(v * 16, 16)]
            ldv = tld[pl.ds(v * 16, 16)]
            mA = ldv < HALF
            permA, totA = _compact(ldv, mA, lanes)
            idsA[pl.ds(cA, 16)] = _vgather(idv, permA)
            mB = (ldv >= HALF) & (ldv < NPW)
            permB, totB = _compact(ldv, mB, lanes)
            idsB[pl.ds(cB, 16)] = _vgather(idv, permB)
            return (cA + totA, cB + totB)
        return lax.fori_loop(0, (cnt + 15) // 16, part_body, (cA, cB))
    cA, cB = lax.fori_loop(0, E // CHUNK, chunk_body, (0, 0))

    pad_id = lanes + (wid * 13 % 256)

    for rnd in range(2):
        lo = lo0 + rnd * HALF
        idsR = idsA if rnd == 0 else idsB
        cR = cA if rnd == 0 else cB
        idsR[pl.ds(cR, 16)] = pad_id
        idsR[pl.ds(cR + 16, 16)] = pad_id
        ng = (cR + GB - 1) // GB

        def init_accs(i, _):
            for k in range(D // 16):
                sl = pl.ds(k * 16, 16)
                sm[i, sl] = jnp.zeros((16,), jnp.float32)
                sq[i, sl] = jnp.zeros((16,), jnp.float32)
                mx[i, sl] = jnp.full((16,), -BIG, jnp.float32)
                mn[i, sl] = jnp.full((16,), BIG, jnp.float32)
            deg[i, pl.ds(0, 16)] = jnp.zeros((16,), jnp.float32)
            return 0
        lax.fori_loop(0, NLOC, init_accs, 0)

        def group_body(g, _):
            for t in range(GB // 16):
                sl = pl.ds(t * 16, 16)
                idg[sl] = idsR[pl.ds(g * GB + t * 16, 16)]
            idg_s = idg.at[pl.ds(0, GB)]
            cpc = pltpu.async_copy(c_hbm.at[idg_s], cbuf.at[0], sem_c)
            cpd = pltpu.async_copy(dst_hbm.at[idg_s],
                                   dstg.at[pl.ds(0, GB)], sem_a2)
            pltpu.async_copy(src_hbm.at[idg_s],
                             srcb.at[pl.ds(0, GB)], sem_a).wait()
            pltpu.async_copy(a_hbm.at[srcb.at[pl.ds(0, GB)]],
                             abuf.at[0], sem_a).wait()
            cpc.wait()
            cpd.wait()
            for t in range(GB // 16):
                lv = dstg[pl.ds(t * 16, 16)] - lo
                lvec = jnp.where((lv >= 0) & (lv < HALF), lv, HALF)
                for e16 in range(16):
                    e = t * 16 + e16
                    l = lvec[e16]
                    deg[l, pl.ds(0, 16)] += ones16
                    for k in range(D // 16):
                        sl = pl.ds(k * 16, 16)
                        u = abuf[0, e, sl] + cbuf[0, e, sl]
                        plsc.addupdate(sm.at[l, sl], u)
                        plsc.addupdate(sq.at[l, sl], u * u)
                        mx[l, sl] = jnp.maximum(mx[l, sl], u)
                        mn[l, sl] = jnp.minimum(mn[l, sl], u)
            return 0
        lax.fori_loop(0, ng, group_body, 0)

        pltpu.sync_copy(sm.at[pl.ds(0, HALF)], sm_hbm.at[pl.ds(lo, HALF)])
        pltpu.sync_copy(sq.at[pl.ds(0, HALF)], sq_hbm.at[pl.ds(lo, HALF)])
        pltpu.sync_copy(mx.at[pl.ds(0, HALF)], mx_hbm.at[pl.ds(lo, HALF)])
        pltpu.sync_copy(mn.at[pl.ds(0, HALF)], mn_hbm.at[pl.ds(lo, HALF)])
        pltpu.sync_copy(deg.at[pl.ds(0, HALF)], deg_hbm.at[pl.ds(lo, HALF)])


def _sc_aggregate(A, C, src, dst):
    mesh = plsc.VectorSubcoreMesh(core_axis_name="c", subcore_axis_name="s")
    f32 = jnp.float32
    kern = functools.partial(
        pl.kernel, mesh=mesh,
        out_type=[jax.ShapeDtypeStruct((NPAD, D), f32),
                  jax.ShapeDtypeStruct((NPAD, D), f32),
                  jax.ShapeDtypeStruct((NPAD, D), f32),
                  jax.ShapeDtypeStruct((NPAD, D), f32),
                  jax.ShapeDtypeStruct((NPAD, 16), f32)],
        scratch_types=[
            pltpu.VMEM((CHUNK,), jnp.int32),
            pltpu.VMEM((CHUNK + 32,), jnp.int32),
            pltpu.VMEM((CHUNK + 32,), jnp.int32),
            pltpu.VMEM((LCAP,), jnp.int32),
            pltpu.VMEM((LCAP,), jnp.int32),
            pltpu.VMEM((2 * GB,), jnp.int32),
            pltpu.VMEM((2 * GB,), jnp.int32),
            pltpu.VMEM((2 * GB,), jnp.int32),
            pltpu.VMEM((2, GB, D), f32),
            pltpu.VMEM((2, GB, D), f32),
            pltpu.VMEM((NLOC, D), f32),
            pltpu.VMEM((NLOC, D), f32),
            pltpu.VMEM((NLOC, D), f32),
            pltpu.VMEM((NLOC, D), f32),
            pltpu.VMEM((NLOC, 16), f32),
            pltpu.SemaphoreType.DMA,
            pltpu.SemaphoreType.DMA,
            pltpu.SemaphoreType.DMA,
            pltpu.SemaphoreType.DMA,
        ],
    )(_sc_agg_body)
    return kern(A, C, src, dst)


def _post_kernel(hc_ref, b_ref, sm_ref, sq_ref, mx_ref, mn_ref, degb_ref,
                 whc_ref, w1_ref, w2_ref, w3_ref, w4_ref, bb_ref,
                 snorm_ref, gam_ref, bet_ref, mu_ref, var_ref,
                 h_ref, p_ref, ho_ref, po_ref):
    B = b_ref[...]
    sumU = sm_ref[...]
    sqU = sq_ref[...]
    deg = degb_ref[...][:, :1]
    safe = jnp.maximum(deg, 1.0)
    mean = (sumU + deg * B) / safe
    meansq = (sqU + 2.0 * B * sumU + deg * (B * B)) / safe
    var = jnp.maximum(meansq - mean * mean, 0.0)
    std = jnp.sqrt(var + EPS)
    pos = deg > 0.0
    mxf = jnp.where(pos, mx_ref[...] + B, 0.0)
    mnf = jnp.where(pos, mn_ref[...] + B, 0.0)
    h3 = jnp.dot(hc_ref[...], whc_ref[...], preferred_element_type=jnp.float32)
    h3 += jnp.dot(mean, w1_ref[...], preferred_element_type=jnp.float32)
    h3 += jnp.dot(mxf, w2_ref[...], preferred_element_type=jnp.float32)
    h3 += jnp.dot(mnf, w3_ref[...], preferred_element_type=jnp.float32)
    h3 += jnp.dot(std, w4_ref[...], preferred_element_type=jnp.float32)
    h3 += bb_ref[...]
    h3 = h3 * snorm_ref[...]
    h3 = (h3 - mu_ref[...]) * lax.rsqrt(var_ref[...] + EPS) * gam_ref[...] + bet_ref[...]
    h3 = jnp.maximum(h3, 0.0)
    ho_ref[...] = h_ref[...] + h3
    po_ref[...] = p_ref[...] + jnp.tanh(h3)


def kernel(h, p, e, snorm_n, edge_index, W_pre_h, b_pre_h, W_pre_p, b_pre_p,
           W_post_h, b_post_h, W_post_p, b_post_p, bn_gamma, bn_beta,
           bn_mean, bn_var):
    hc = jnp.concatenate([h, p], axis=1)
    W1 = W_pre_h[:2 * D]
    W2 = W_pre_h[2 * D:4 * D]
    W3 = W_pre_h[4 * D:]

    A, B = pl.pallas_call(
        _pre_ab_kernel,
        out_shape=(jax.ShapeDtypeStruct((N, D), jnp.float32),
                   jax.ShapeDtypeStruct((N, D), jnp.float32)),
        grid=(10,),
        in_specs=[pl.BlockSpec((N // 10, 2 * D), lambda i: (i, 0)),
                  pl.BlockSpec((2 * D, D), lambda i: (0, 0)),
                  pl.BlockSpec((2 * D, D), lambda i: (0, 0))],
        out_specs=(pl.BlockSpec((N // 10, D), lambda i: (i, 0)),
                   pl.BlockSpec((N // 10, D), lambda i: (i, 0))),
    )(hc, W1, W2)

    C = pl.pallas_call(
        _edge_c_kernel,
        out_shape=jax.ShapeDtypeStruct((E, D), jnp.float32),
        grid=(16,),
        in_specs=[pl.BlockSpec((E // 16, ED), lambda i: (i, 0)),
                  pl.BlockSpec((ED, D), lambda i: (0, 0)),
                  pl.BlockSpec((D,), lambda i: (0,))],
        out_specs=pl.BlockSpec((E // 16, D), lambda i: (i, 0)),
    )(e, W3, b_pre_h)

    src = edge_index[0]
    dst = edge_index[1]
    sm, sq, mx, mn, degb = _sc_aggregate(A, C, src, dst)

    ho, po = pl.pallas_call(
        _post_kernel,
        out_shape=(jax.ShapeDtypeStruct((N, D), jnp.float32),
                   jax.ShapeDtypeStruct((N, D), jnp.float32)),
        grid=(10,),
        in_specs=[pl.BlockSpec((N // 10, 2 * D), lambda i: (i, 0)),
                  pl.BlockSpec((N // 10, D), lambda i: (i, 0)),
                  pl.BlockSpec((N // 10, D), lambda i: (i, 0)),
                  pl.BlockSpec((N // 10, D), lambda i: (i, 0)),
                  pl.BlockSpec((N // 10, D), lambda i: (i, 0)),
                  pl.BlockSpec((N // 10, D), lambda i: (i, 0)),
                  pl.BlockSpec((N // 10, 16), lambda i: (i, 0)),
                  pl.BlockSpec((2 * D, D), lambda i: (0, 0)),
                  pl.BlockSpec((D, D), lambda i: (0, 0)),
                  pl.BlockSpec((D, D), lambda i: (0, 0)),
                  pl.BlockSpec((D, D), lambda i: (0, 0)),
                  pl.BlockSpec((D, D), lambda i: (0, 0)),
                  pl.BlockSpec((D,), lambda i: (0,)),
                  pl.BlockSpec((N // 10, 1), lambda i: (i, 0)),
                  pl.BlockSpec((D,), lambda i: (0,)),
                  pl.BlockSpec((D,), lambda i: (0,)),
                  pl.BlockSpec((D,), lambda i: (0,)),
                  pl.BlockSpec((D,), lambda i: (0,)),
                  pl.BlockSpec((N // 10, D), lambda i: (i, 0)),
                  pl.BlockSpec((N // 10, D), lambda i: (i, 0))],
        out_specs=(pl.BlockSpec((N // 10, D), lambda i: (i, 0)),
                   pl.BlockSpec((N // 10, D), lambda i: (i, 0))),
    )(hc, B, sm[:N], sq[:N], mx[:N], mn[:N], degb[:N],
      W_post_h[:2 * D], W_post_h[2 * D:3 * D], W_post_h[3 * D:4 * D],
      W_post_h[4 * D:5 * D], W_post_h[5 * D:], b_post_h, snorm_n,
      bn_gamma, bn_beta, bn_mean, bn_var, h, p)
    return (ho, po)
